# baseline (speedup 1.0000x reference)
import functools

import jax
import jax.numpy as jnp
import numpy as np
from jax import lax
from jax.experimental import pallas as pl
from jax.experimental.pallas import tpu as pltpu
from jax.experimental.pallas import tpu_sc as plsc

N_NODES = 10000
N_EDGES = 320000
NUM_CLASSES = 16
HIDDEN = 128
ATTR = 4
N_LAYERS = 4
NUM_GRAPHS = 64
OUT = 1

NC, NS = 2, 16
NW = NC * NS
CHUNK = 128
N_CHUNKS = N_EDGES // CHUNK
WORK_ITERS = (N_CHUNKS + NW - 1) // NW

_PREC = lax.Precision.HIGHEST
_F32 = jnp.float32


def _dot(a, b):
    return jnp.dot(a, b, precision=_PREC, preferred_element_type=_F32)


def _bdot(a, b):
    return jnp.dot(a.astype(jnp.bfloat16), b.astype(jnp.bfloat16),
                   preferred_element_type=_F32)


def _silu(v):
    return v * (1.0 / (1.0 + jnp.exp(-v)))


@functools.cache
def _make_sc_gather():
    D = HIDDEN
    mesh = plsc.VectorSubcoreMesh(core_axis_name="c", subcore_axis_name="s",
                                  num_cores=NC, num_subcores=NS)

    @functools.partial(
        pl.kernel,
        out_type=[
            jax.ShapeDtypeStruct((N_EDGES, D), _F32),
            jax.ShapeDtypeStruct((N_EDGES, D), _F32),
        ],
        mesh=mesh,
        scratch_types=[
            pltpu.VMEM((CHUNK,), jnp.int32),
            pltpu.VMEM((CHUNK,), jnp.int32),
            pltpu.VMEM((CHUNK, D), _F32),
            pltpu.VMEM((CHUNK, D), _F32),
            pltpu.SemaphoreType.DMA,
            pltpu.SemaphoreType.DMA,
        ],
    )
    def gather_k(dst_hbm, src_hbm, tab_hbm, outd_hbm, outs_hbm,
                 idx_d, idx_s, rows_d, rows_s, sem_d, sem_s):
        wid = lax.axis_index("s") * NC + lax.axis_index("c")

        def step(t, carry):
            chunk = wid + t * NW

            @pl.when(chunk < N_CHUNKS)
            def _():
                base = chunk * CHUNK
                pltpu.sync_copy(dst_hbm.at[pl.ds(base, CHUNK)], idx_d)
                pltpu.sync_copy(src_hbm.at[pl.ds(base, CHUNK)], idx_s)
                cpd = pltpu.async_copy(tab_hbm.at[idx_d], rows_d, sem_d)
                cps = pltpu.async_copy(tab_hbm.at[idx_s], rows_s, sem_s)
                cpd.wait()
                cps.wait()
                pltpu.sync_copy(rows_d, outd_hbm.at[pl.ds(base, CHUNK)])
                pltpu.sync_copy(rows_s, outs_hbm.at[pl.ds(base, CHUNK)])

            return carry

        lax.fori_loop(0, WORK_ITERS, step, 0)

    return gather_k


@functools.cache
def _make_sc_scatter():
    D = HIDDEN
    mesh = plsc.VectorSubcoreMesh(core_axis_name="c", subcore_axis_name="s",
                                  num_cores=NC, num_subcores=NS)
    rows_per = 640
    last_rows = N_NODES - rows_per * (NS - 1)

    @functools.partial(
        pl.kernel,
        out_type=jax.ShapeDtypeStruct((NC, N_NODES, D), _F32),
        mesh=mesh,
        scratch_types=[
            pltpu.VMEM((1, CHUNK), jnp.int32),
            pltpu.VMEM((CHUNK, D), _F32),
            pltpu.VMEM_SHARED((N_NODES, D), _F32),
        ],
    )
    def scatter_k(dst_hbm, vals_hbm, zeros_hbm, out_hbm, idx_v, rows_v, acc):
        cid = lax.axis_index("c")
        sid = lax.axis_index("s")
        wid = sid * NC + cid

        @pl.when(sid < NS - 1)
        def _():
            pltpu.sync_copy(zeros_hbm.at[pl.ds(sid * rows_per, rows_per)],
                            acc.at[pl.ds(sid * rows_per, rows_per)])

        @pl.when(sid == NS - 1)
        def _():
            pltpu.sync_copy(zeros_hbm.at[pl.ds(sid * rows_per, last_rows)],
                            acc.at[pl.ds(sid * rows_per, last_rows)])

        plsc.subcore_barrier()

        def step(t, carry):
            chunk = wid + t * NW

            @pl.when(chunk < N_CHUNKS)
            def _():
                base = chunk * CHUNK
                pltpu.sync_copy(dst_hbm.at[pl.ds(base, CHUNK)], idx_v.at[0])
                pltpu.sync_copy(vals_hbm.at[pl.ds(base, CHUNK)], rows_v)
                pltpu.sync_copy(rows_v, acc.at[idx_v.at[0]], add=True)

            return carry

        lax.fori_loop(0, WORK_ITERS, step, 0)
        plsc.subcore_barrier()

        @pl.when(sid < NS - 1)
        def _():
            pltpu.sync_copy(acc.at[pl.ds(sid * rows_per, rows_per)],
                            out_hbm.at[cid, pl.ds(sid * rows_per, rows_per)])

        @pl.when(sid == NS - 1)
        def _():
            pltpu.sync_copy(acc.at[pl.ds(sid * rows_per, last_rows)],
                            out_hbm.at[cid, pl.ds(sid * rows_per, last_rows)])

    return scatter_k


_RE = 1600
_GE = N_EDGES // _RE
_RN = 2000
_GN = N_NODES // _RN


@functools.cache
def _make_tc_dot(M, R, K, use_silu, residual):
    grid = (M // R,)

    def body(*refs):
        if residual:
            f_ref, w_ref, res_ref, o_ref = refs
        else:
            f_ref, w_ref, o_ref = refs
        z = jnp.dot(f_ref[...], w_ref[...].astype(jnp.bfloat16),
                    preferred_element_type=_F32)
        if use_silu:
            z = _silu(z)
        if residual:
            z = res_ref[...] + z
        o_ref[...] = z

    in_specs = [pl.BlockSpec((R, K), lambda i: (i, 0)),
                pl.BlockSpec((K, HIDDEN), lambda i: (0, 0))]
    if residual:
        in_specs.append(pl.BlockSpec((R, HIDDEN), lambda i: (i, 0)))

    def call(f, W, res=None):
        args = (f, W) if res is None else (f, W, res)
        return pl.pallas_call(
            body, grid=grid, in_specs=in_specs,
            out_specs=pl.BlockSpec((R, HIDDEN), lambda i: (i, 0)),
            out_shape=jax.ShapeDtypeStruct((M, HIDDEN), _F32))(*args)

    return call


def _tp_bf16(h, attr):
    f = (h[:, :, None] * attr[:, None, :]).reshape(h.shape[0], -1)
    return f.astype(jnp.bfloat16)


def _head_body(hp_ref, b_ref, Wp2_ref, Wq1_ref, Wq2_ref, out_ref, pooled, cnt):
    i = pl.program_id(0)

    @pl.when(i == 0)
    def _():
        pooled[...] = jnp.zeros_like(pooled)
        cnt[...] = jnp.zeros_like(cnt)

    hp2 = _bdot(hp_ref[...], Wp2_ref[...])
    oneB = (b_ref[...] == lax.broadcasted_iota(jnp.int32, (_RN, NUM_GRAPHS), 1)
            ).astype(_F32)
    pooled[...] += lax.dot_general(oneB, hp2, (((0,), (0,)), ((), ())),
                                   precision=_PREC, preferred_element_type=_F32)
    cnt[...] += lax.dot_general(oneB, jnp.ones((_RN, 1), _F32),
                                (((0,), (0,)), ((), ())),
                                precision=_PREC, preferred_element_type=_F32)

    @pl.when(i == _GN - 1)
    def _():
        pm = pooled[...] / jnp.maximum(cnt[...], 1.0)
        t = _silu(_bdot(pm, Wq1_ref[...]))
        out_ref[...] = _bdot(t, Wq2_ref[...])


def _tc_head(hp, batch2d, Wp2, Wq1, Wq2):
    return pl.pallas_call(
        _head_body,
        grid=(_GN,),
        in_specs=[pl.BlockSpec((_RN, HIDDEN), lambda i: (i, 0)),
                  pl.BlockSpec((_RN, 1), lambda i: (i, 0)),
                  pl.BlockSpec((HIDDEN, HIDDEN), lambda i: (0, 0)),
                  pl.BlockSpec((HIDDEN, HIDDEN), lambda i: (0, 0)),
                  pl.BlockSpec((HIDDEN, OUT), lambda i: (0, 0))],
        out_specs=pl.BlockSpec((NUM_GRAPHS, OUT), lambda i: (0, 0)),
        out_shape=jax.ShapeDtypeStruct((NUM_GRAPHS, OUT), _F32),
        scratch_shapes=[pltpu.VMEM((NUM_GRAPHS, HIDDEN), _F32),
                        pltpu.VMEM((NUM_GRAPHS, 1), _F32)],
    )(hp, batch2d, Wp2, Wq1, Wq2)


def kernel(x, pos, edge_index, batch, We1, We2, Wm1, Wm2, Wu1, Wu2,
           Wp1, Wp2, Wq1, Wq2):
    src = edge_index[0]
    dst = edge_index[1]
    pos_pad = jnp.pad(pos.astype(_F32), ((0, 0), (0, HIDDEN - 3)))
    x2d = x[:, None]
    batch2d = batch[:, None]

    zeros128 = jnp.zeros((N_NODES, HIDDEN), _F32)

    posd, poss = _make_sc_gather()(dst, src, pos_pad)
    rel = poss[:, 0:3] - posd[:, 0:3]
    edge_dist = jnp.sum(rel * rel, axis=-1, keepdims=True)
    r = jnp.sqrt(edge_dist + 1e-12)
    u = rel / r
    s3 = float(np.sqrt(3.0))
    ea4 = jnp.concatenate([jnp.ones_like(r), s3 * u[:, 1:2], s3 * u[:, 2:3],
                           s3 * u[:, 0:1]], axis=-1)
    cnt = jnp.zeros((N_NODES, 1), _F32).at[dst].add(1.0)
    na = jnp.zeros((N_NODES, ATTR), _F32).at[dst].add(ea4) / jnp.maximum(cnt, 1.0)
    na = na.at[:, 0].set(1.0)

    dot_n1 = _make_tc_dot(N_NODES, _RN, ATTR * NUM_CLASSES, True, False)
    dot_n2 = _make_tc_dot(N_NODES, _RN, ATTR * HIDDEN, False, False)
    dot_u1 = _make_tc_dot(N_NODES, _RN, 2 * ATTR * HIDDEN, True, False)
    dot_u2 = _make_tc_dot(N_NODES, _RN, ATTR * HIDDEN, False, True)
    dot_p = _make_tc_dot(N_NODES, _RN, ATTR * HIDDEN, True, False)
    dot_e1 = _make_tc_dot(N_EDGES, _RE, (2 * HIDDEN + 1) * ATTR, True, False)
    dot_e2 = _make_tc_dot(N_EDGES, _RE, ATTR * HIDDEN, True, False)

    onehot = jax.nn.one_hot(x, NUM_CLASSES, dtype=_F32)
    h = dot_n1(_tp_bf16(onehot, na), We1)
    h = dot_n2(_tp_bf16(h, na), We2)

    for i in range(N_LAYERS):
        hd, hs = _make_sc_gather()(dst, src, h)
        m_in = jnp.concatenate([hd, hs, edge_dist], axis=-1)
        m1 = dot_e1(_tp_bf16(m_in, ea4), Wm1[i])
        m2 = dot_e2(_tp_bf16(m1, ea4), Wm2[i])
        agg2 = _make_sc_scatter()(dst, m2, zeros128)
        agg = agg2[0] + agg2[1]
        cat = jnp.concatenate([h, agg], axis=-1)
        uu = dot_u1(_tp_bf16(cat, na), Wu1[i])
        h = dot_u2(_tp_bf16(uu, na), Wu2[i], h)

    hp = dot_p(_tp_bf16(h, na), Wp1)
    return _tc_head(hp, batch2d, Wp2, Wq1, Wq2)
